# baseline (speedup 1.0000x reference)
import jax
import jax.numpy as jnp
from jax.experimental import pallas as pl
from jax.experimental.pallas import tpu as pltpu

B, S, D = 2, 2048, 768
E, K, H = 8, 2, 768
T = B * S
JB = 1024
NJ = T // JB
EPAD = 128

_NEG = -1e30


def _moe_kernel(x_ref, gw_ref, gb_ref, w1_ref, b1_ref, w2_ref, b2_ref,
                wo_ref, bo_ref, out_ref, acc_ref, probs_ref):
    e = pl.program_id(0)
    j = pl.program_id(1)
    rows = pl.ds(j * JB, JB)
    lane = jax.lax.broadcasted_iota(jnp.int32, (JB, EPAD), 1)

    @pl.when(e == 0)
    def _gate():
        xb = x_ref[rows, :]
        logits = jnp.dot(xb, gw_ref[...], preferred_element_type=jnp.float32)
        logits = logits + gb_ref[...]

        big = jnp.int32(EPAD)
        l1 = jnp.max(logits, axis=-1, keepdims=True)
        i1 = jnp.min(jnp.where(logits == l1, lane, big), axis=-1,
                     keepdims=True)
        oh1 = (lane == i1)
        masked = jnp.where(oh1, _NEG, logits)
        l2 = jnp.max(masked, axis=-1, keepdims=True)
        i2 = jnp.min(jnp.where(masked == l2, lane, big), axis=-1,
                     keepdims=True)
        oh2 = (lane == i2)

        r = jnp.exp(l2 - l1)
        w_top1 = 1.0 / (1.0 + r)
        probs_ref[rows, :] = (w_top1 * oh1.astype(jnp.float32)
                              + (1.0 - w_top1) * oh2.astype(jnp.float32))

    @pl.when(e < E)
    def _ffn():
        xb = x_ref[rows, :]
        h = jnp.dot(xb, w1_ref[0], preferred_element_type=jnp.float32)
        h = jnp.maximum(h + b1_ref[0], 0.0)
        y = jnp.dot(h, w2_ref[0], preferred_element_type=jnp.float32)
        y = y + b2_ref[0]
        scale = jnp.sum(probs_ref[rows, :] * (lane == e).astype(jnp.float32),
                        axis=-1, keepdims=True)
        contrib = y * scale

        @pl.when(e == 0)
        def _init():
            acc_ref[rows, :] = contrib

        @pl.when(e > 0)
        def _accum():
            acc_ref[rows, :] = acc_ref[rows, :] + contrib

    @pl.when(e == E)
    def _proj():
        out = jnp.dot(acc_ref[rows, :], wo_ref[...],
                      preferred_element_type=jnp.float32)
        out_ref[...] = out + bo_ref[...]


@jax.jit
def kernel(x, gate_W, gate_b, W1, b1, W2, b2, Wout, bout):
    xf = x.reshape(T, D)
    gw = jnp.pad(gate_W, ((0, 0), (0, EPAD - E)))
    gb = jnp.full((1, EPAD), _NEG, dtype=jnp.float32).at[0, :E].set(gate_b)

    grid = (E + 1, NJ)
    res = lambda shape: pl.BlockSpec(shape, lambda e, j: (0,) * len(shape))
    eidx = lambda nd: pl.BlockSpec(
        (1,) + nd, lambda e, j: (jnp.minimum(e, E - 1),) + (0,) * len(nd))
    out = pl.pallas_call(
        _moe_kernel,
        grid=grid,
        in_specs=[
            res((T, D)),
            res((D, EPAD)),
            res((1, EPAD)),
            eidx((D, H)),
            eidx((1, H)),
            eidx((H, D)),
            eidx((1, D)),
            res((D, D)),
            res((1, D)),
        ],
        out_specs=pl.BlockSpec(
            (JB, D), lambda e, j: (jnp.where(e == E, j, 0), 0)),
        out_shape=jax.ShapeDtypeStruct((T, D), jnp.float32),
        scratch_shapes=[
            pltpu.VMEM((T, D), jnp.float32),
            pltpu.VMEM((T, EPAD), jnp.float32),
        ],
        compiler_params=pltpu.CompilerParams(
            vmem_limit_bytes=120 * 1024 * 1024,
        ),
    )(xf, gw, gb, W1, b1.reshape(E, 1, H), W2, b2.reshape(E, 1, D),
      Wout, bout.reshape(1, D))
    return out.reshape(B, S, D)


# speedup vs baseline: 1.1945x; 1.1945x over previous
import jax
import jax.numpy as jnp
from jax.experimental import pallas as pl
from jax.experimental.pallas import tpu as pltpu

B, S, D = 2, 2048, 768
E, K, H = 8, 2, 768
T = B * S
TB = 1024
EPAD = 128

_NEG = -1e30


def _moe_kernel(x_ref, gw_ref, gb_ref, w1_ref, b1_ref, w2_ref, b2_ref,
                wo_ref, bo_ref, out_ref):
    xb = x_ref[...]

    logits = jnp.dot(xb, gw_ref[...], preferred_element_type=jnp.float32)
    logits = logits + gb_ref[...]

    lane = jax.lax.broadcasted_iota(jnp.int32, (TB, EPAD), 1)
    big = jnp.int32(EPAD)

    l1 = jnp.max(logits, axis=-1, keepdims=True)
    i1 = jnp.min(jnp.where(logits == l1, lane, big), axis=-1, keepdims=True)
    oh1 = (lane == i1)
    masked = jnp.where(oh1, _NEG, logits)
    l2 = jnp.max(masked, axis=-1, keepdims=True)
    i2 = jnp.min(jnp.where(masked == l2, lane, big), axis=-1, keepdims=True)
    oh2 = (lane == i2)

    r = jnp.exp(l2 - l1)
    w_top1 = 1.0 / (1.0 + r)
    w_top2 = 1.0 - w_top1
    probs = w_top1 * oh1.astype(jnp.float32) + w_top2 * oh2.astype(jnp.float32)

    acc = jnp.zeros((TB, D), dtype=jnp.float32)
    for e in range(E):
        h = jnp.dot(xb, w1_ref[e], preferred_element_type=jnp.float32)
        h = jnp.maximum(h + b1_ref[e][None, :], 0.0)
        y = jnp.dot(h, w2_ref[e], preferred_element_type=jnp.float32)
        y = y + b2_ref[e][None, :]
        acc = acc + y * probs[:, e][:, None]

    out = jnp.dot(acc, wo_ref[...], preferred_element_type=jnp.float32)
    out_ref[...] = out + bo_ref[...]


@jax.jit
def kernel(x, gate_W, gate_b, W1, b1, W2, b2, Wout, bout):
    xf = x.reshape(T, D)
    gw = jnp.pad(gate_W, ((0, 0), (0, EPAD - E)))
    gb = jnp.full((1, EPAD), _NEG, dtype=jnp.float32).at[0, :E].set(gate_b)

    grid = (T // TB,)
    full = lambda shape: pl.BlockSpec(shape, lambda i: (0,) * len(shape))
    out = pl.pallas_call(
        _moe_kernel,
        grid=grid,
        in_specs=[
            pl.BlockSpec((TB, D), lambda i: (i, 0)),
            full((D, EPAD)),
            full((1, EPAD)),
            full((E, D, H)),
            full((E, H)),
            full((E, H, D)),
            full((E, D)),
            full((D, D)),
            full((1, D)),
        ],
        out_specs=pl.BlockSpec((TB, D), lambda i: (i, 0)),
        out_shape=jax.ShapeDtypeStruct((T, D), jnp.float32),
        compiler_params=pltpu.CompilerParams(
            vmem_limit_bytes=120 * 1024 * 1024,
        ),
    )(xf, gw, gb, W1, b1, W2, b2, Wout, bout.reshape(1, D))
    return out.reshape(B, S, D)
